# baseline (speedup 1.0000x reference)
import functools

import jax
import jax.numpy as jnp
from jax import lax
from jax.experimental import pallas as pl
from jax.experimental.pallas import tpu as pltpu
from jax.experimental.pallas import tpu_sc as plsc

D = 64
NC = 2
NS = 16
NW = NC * NS
LANES = 16
CHUNK = 512
IDX_ROW = 128


def _sc_scores_body(n_pairs, b_pos, u_w, v_w, idx_u, idx_v, out,
                    idx_u_v, idx_v_v, u_rows, v_rows, scores_v, part_v, sem):
    wid = lax.axis_index("s") * NC + lax.axis_index("c")
    pairs_per_w = n_pairs // NW
    n_chunks = pairs_per_w // CHUNK
    rows_per_chunk = CHUNK // IDX_ROW
    base_pair = wid * pairs_per_w
    base_row = wid * (pairs_per_w // IDX_ROW)
    lane = lax.iota(jnp.int32, LANES)

    for c in range(n_chunks):
        row0 = base_row + c * rows_per_chunk
        pltpu.sync_copy(idx_u.at[pl.ds(row0, rows_per_chunk)], idx_u_v)
        pltpu.sync_copy(idx_v.at[pl.ds(row0, rows_per_chunk)], idx_v_v)
        copies = []
        for j in range(rows_per_chunk):
            copies.append(pltpu.async_copy(
                u_w.at[idx_u_v.at[j]], u_rows.at[pl.ds(j * IDX_ROW, IDX_ROW)], sem))
            copies.append(pltpu.async_copy(
                v_w.at[idx_v_v.at[j]], v_rows.at[pl.ds(j * IDX_ROW, IDX_ROW)], sem))
        for cp in copies:
            cp.wait()

        chunk_pos0 = base_pair + c * CHUNK

        def group(g, _):
            for p in range(LANES):
                row = g * LANES + p
                part = jnp.zeros((LANES,), jnp.float32)
                for k in range(D // LANES):
                    part = part + (u_rows[row, pl.ds(k * LANES, LANES)]
                                   * v_rows[row, pl.ds(k * LANES, LANES)])
                part_v[pl.ds(p * LANES, LANES)] = part
            acc = jnp.zeros((LANES,), jnp.float32)
            col0 = lane * LANES
            for j in range(LANES):
                acc = acc + plsc.load_gather(part_v, [col0 + j])
            gpos = chunk_pos0 + g * LANES + lane
            sign = jnp.where(gpos < b_pos, 1.0, -1.0).astype(jnp.float32)
            scores_v[pl.ds(g * LANES, LANES)] = acc * sign
            return _

        lax.fori_loop(0, CHUNK // LANES, group, 0)
        pltpu.sync_copy(scores_v, out.at[pl.ds(chunk_pos0, CHUNK)])


def _sc_scores(u_w, v_w, idx_u, idx_v, n_pairs, b_pos):
    mesh = plsc.VectorSubcoreMesh(core_axis_name="c", subcore_axis_name="s")
    body = functools.partial(_sc_scores_body, n_pairs, b_pos)
    return pl.kernel(
        body,
        out_type=jax.ShapeDtypeStruct((n_pairs,), jnp.float32),
        mesh=mesh,
        scratch_types=[
            pltpu.VMEM((CHUNK // IDX_ROW, IDX_ROW), jnp.int32),
            pltpu.VMEM((CHUNK // IDX_ROW, IDX_ROW), jnp.int32),
            pltpu.VMEM((CHUNK, D), jnp.float32),
            pltpu.VMEM((CHUNK, D), jnp.float32),
            pltpu.VMEM((CHUNK,), jnp.float32),
            pltpu.VMEM((LANES * LANES,), jnp.float32),
            pltpu.SemaphoreType.DMA,
        ],
        compiler_params=pltpu.CompilerParams(
            needs_layout_passes=False, use_tc_tiling_on_sc=False),
    )(u_w, v_w, idx_u, idx_v)


def _tc_loss_body(s_ref, o_ref):
    x = s_ref[:]
    o_ref[0, 0] = -jnp.sum(jax.nn.log_sigmoid(x))


def _tc_loss(scores2d):
    out = pl.pallas_call(
        _tc_loss_body,
        out_shape=jax.ShapeDtypeStruct((1, 1), jnp.float32),
        in_specs=[pl.BlockSpec(memory_space=pltpu.VMEM)],
        out_specs=pl.BlockSpec(memory_space=pltpu.SMEM),
    )(scores2d)
    return out[0, 0]


def kernel(pos_u, pos_v, neg_u, neg_v, u_weight, v_weight):
    b_pos = pos_u.shape[0]
    n_pairs = b_pos + neg_u.shape[0]
    idx_u = jnp.concatenate([pos_u, neg_u]).reshape(n_pairs // IDX_ROW, IDX_ROW)
    idx_v = jnp.concatenate([pos_v, neg_v]).reshape(n_pairs // IDX_ROW, IDX_ROW)
    scores = _sc_scores(u_weight, v_weight, idx_u, idx_v, n_pairs, b_pos)
    return _tc_loss(scores.reshape(n_pairs // IDX_ROW, IDX_ROW))


# speedup vs baseline: 1.8357x; 1.8357x over previous
import functools

import jax
import jax.numpy as jnp
from jax import lax
from jax.experimental import pallas as pl
from jax.experimental.pallas import tpu as pltpu
from jax.experimental.pallas import tpu_sc as plsc

VOCAB_PAD = 2048
D = 64
NC = 2
NS = 16
NW = NC * NS
LANES = 16
CHUNK = 512
IDX_ROW = 128


def _relayout_body(u1, u2, v1, v2, ou, ov):
    ou[:] = jnp.concatenate([u1[:], u2[:]], axis=0).T
    ov[:] = jnp.concatenate([v1[:], v2[:]], axis=0).T


def _relayout(u_t, v_t):
    vocab = u_t.shape[1]
    n_blk = (vocab + VOCAB_PAD - 1) // VOCAB_PAD
    max_col_blk = (vocab + 1023) // 1024 - 1
    lo = lambda i: (0, jnp.minimum(2 * i, max_col_blk))
    hi = lambda i: (0, jnp.minimum(2 * i + 1, max_col_blk))
    out_shape = jax.ShapeDtypeStruct((n_blk * 1024, 128), jnp.float32)
    return pl.pallas_call(
        _relayout_body,
        grid=(n_blk,),
        in_specs=[
            pl.BlockSpec((D, 1024), lo),
            pl.BlockSpec((D, 1024), hi),
            pl.BlockSpec((D, 1024), lo),
            pl.BlockSpec((D, 1024), hi),
        ],
        out_specs=[
            pl.BlockSpec((1024, 128), lambda i: (i, 0)),
            pl.BlockSpec((1024, 128), lambda i: (i, 0)),
        ],
        out_shape=[out_shape, out_shape],
    )(u_t, u_t, v_t, v_t)


def _remap(v):
    return ((v & -2048) + ((v & 1023) << 1)) + ((v >> 10) & 1)


def _sc_scores_body(n_pairs, b_pos, u_w, v_w, idx_u, idx_v, out,
                    idx_u_v, idx_v_v, u_rows, v_rows, scores_v, part_v, sem):
    wid = lax.axis_index("s") * NC + lax.axis_index("c")
    pairs_per_w = n_pairs // NW
    n_chunks = pairs_per_w // CHUNK
    rows_per_chunk = CHUNK // IDX_ROW
    base_pair = wid * pairs_per_w
    base_row = wid * (pairs_per_w // IDX_ROW)
    lane = lax.iota(jnp.int32, LANES)

    for c in range(n_chunks):
        row0 = base_row + c * rows_per_chunk
        pltpu.sync_copy(idx_u.at[pl.ds(row0, rows_per_chunk)], idx_u_v)
        pltpu.sync_copy(idx_v.at[pl.ds(row0, rows_per_chunk)], idx_v_v)
        for r in range(rows_per_chunk):
            for q in range(IDX_ROW // LANES):
                sl = pl.ds(q * LANES, LANES)
                idx_u_v[r, sl] = _remap(idx_u_v[r, sl])
                idx_v_v[r, sl] = _remap(idx_v_v[r, sl])
        copies = []
        for j in range(rows_per_chunk):
            copies.append(pltpu.async_copy(
                u_w.at[idx_u_v.at[j]], u_rows.at[pl.ds(j * IDX_ROW, IDX_ROW)], sem))
            copies.append(pltpu.async_copy(
                v_w.at[idx_v_v.at[j]], v_rows.at[pl.ds(j * IDX_ROW, IDX_ROW)], sem))
        for cp in copies:
            cp.wait()

        chunk_pos0 = base_pair + c * CHUNK

        def group(g, _):
            for p in range(LANES):
                row = g * LANES + p
                part = jnp.zeros((LANES,), jnp.float32)
                for k in range(D // LANES):
                    part = part + (u_rows[row, pl.ds(k * LANES, LANES)]
                                   * v_rows[row, pl.ds(k * LANES, LANES)])
                part_v[pl.ds(p * LANES, LANES)] = part
            acc = jnp.zeros((LANES,), jnp.float32)
            col0 = lane * LANES
            for j in range(LANES):
                acc = acc + plsc.load_gather(part_v, [col0 + j])
            gpos = chunk_pos0 + g * LANES + lane
            sign = jnp.where(gpos < b_pos, 1.0, -1.0).astype(jnp.float32)
            scores_v[pl.ds(g * LANES, LANES)] = acc * sign
            return _

        lax.fori_loop(0, CHUNK // LANES, group, 0)
        pltpu.sync_copy(scores_v, out.at[pl.ds(chunk_pos0, CHUNK)])


def _sc_scores(u_w, v_w, idx_u, idx_v, n_pairs, b_pos):
    mesh = plsc.VectorSubcoreMesh(core_axis_name="c", subcore_axis_name="s")
    body = functools.partial(_sc_scores_body, n_pairs, b_pos)
    return pl.kernel(
        body,
        out_type=jax.ShapeDtypeStruct((n_pairs,), jnp.float32),
        mesh=mesh,
        scratch_types=[
            pltpu.VMEM((CHUNK // IDX_ROW, IDX_ROW), jnp.int32),
            pltpu.VMEM((CHUNK // IDX_ROW, IDX_ROW), jnp.int32),
            pltpu.VMEM((CHUNK, D), jnp.float32),
            pltpu.VMEM((CHUNK, D), jnp.float32),
            pltpu.VMEM((CHUNK,), jnp.float32),
            pltpu.VMEM((LANES * LANES,), jnp.float32),
            pltpu.SemaphoreType.DMA,
        ],
        compiler_params=pltpu.CompilerParams(
            needs_layout_passes=False, use_tc_tiling_on_sc=False),
    )(u_w, v_w, idx_u, idx_v)


def _tc_loss_body(s_ref, o_ref):
    x = s_ref[:]
    o_ref[0, 0] = -jnp.sum(jax.nn.log_sigmoid(x))


def _tc_loss(scores2d):
    out = pl.pallas_call(
        _tc_loss_body,
        out_shape=jax.ShapeDtypeStruct((1, 1), jnp.float32),
        in_specs=[pl.BlockSpec(memory_space=pltpu.VMEM)],
        out_specs=pl.BlockSpec(memory_space=pltpu.SMEM),
    )(scores2d)
    return out[0, 0]


def kernel(pos_u, pos_v, neg_u, neg_v, u_weight, v_weight):
    b_pos = pos_u.shape[0]
    n_pairs = b_pos + neg_u.shape[0]
    idx_u = jnp.concatenate([pos_u, neg_u]).reshape(n_pairs // IDX_ROW, IDX_ROW)
    idx_v = jnp.concatenate([pos_v, neg_v]).reshape(n_pairs // IDX_ROW, IDX_ROW)
    u2, v2 = _relayout(u_weight.T, v_weight.T)
    u2 = u2.reshape(u2.shape[0] * 2, D)
    v2 = v2.reshape(v2.shape[0] * 2, D)
    scores = _sc_scores(u2, v2, idx_u, idx_v, n_pairs, b_pos)
    return _tc_loss(scores.reshape(n_pairs // IDX_ROW, IDX_ROW))
